# baseline (speedup 1.0000x reference)
import functools

import jax
import jax.numpy as jnp
from jax import lax
from jax.experimental import pallas as pl
from jax.experimental.pallas import tpu as pltpu
from jax.experimental.pallas import tpu_sc as plsc

DIM = 128
QUEUE_SIZE = 65536
BATCH_COLS = 4096

_NW = 32
_COLS_W = QUEUE_SIZE // _NW
_CW = 256
_NCH = _COLS_W // _CW
_NBUF = 3


def _sc_body(lk_ref, q_ref, out_ref, buf, rsem, wsem):
    cid = lax.axis_index("c")
    sid = lax.axis_index("s")
    wid = sid * 2 + cid
    col0 = wid * _COLS_W

    def ring(src_ref):
        def rd(ch):
            b = ch % _NBUF
            return pltpu.make_async_copy(
                src_ref.at[:, pl.ds(col0 + ch * _CW, _CW)], buf.at[b], rsem.at[b]
            )

        def wr(ch):
            b = ch % _NBUF
            return pltpu.make_async_copy(
                buf.at[b], out_ref.at[:, pl.ds(col0 + ch * _CW, _CW)], wsem.at[b]
            )

        for ch in range(_NBUF):
            rd(ch).start()
        for ch in range(_NCH):
            rd(ch).wait()
            wr(ch).start()
            if ch + _NBUF < _NCH:
                wr(ch).wait()
                rd(ch + _NBUF).start()
        for ch in range(max(_NCH - _NBUF, 0), _NCH):
            wr(ch).wait()

    @pl.when(wid < BATCH_COLS // _COLS_W)
    def _():
        ring(lk_ref)

    @pl.when(wid >= BATCH_COLS // _COLS_W)
    def _():
        ring(q_ref)


def kernel(last_k, moco_queue):
    mesh = plsc.VectorSubcoreMesh(core_axis_name="c", subcore_axis_name="s")
    run = functools.partial(
        pl.kernel,
        out_type=jax.ShapeDtypeStruct((DIM, QUEUE_SIZE), jnp.float32),
        mesh=mesh,
        scratch_types=[
            pltpu.VMEM((_NBUF, DIM, _CW), jnp.float32),
            pltpu.SemaphoreType.DMA((_NBUF,)),
            pltpu.SemaphoreType.DMA((_NBUF,)),
        ],
    )(_sc_body)
    return run(last_k, moco_queue)
